# Initial kernel scaffold; baseline (speedup 1.0000x reference)
import jax
import jax.numpy as jnp
from jax.experimental import pallas as pl


def kernel(word_inputs, feature_input_0, feature_input_1, word_emb_table, feat_table_0, feat_table_1):
    raise NotImplementedError("write your pallas kernel here")


# speedup vs baseline: 2.3247x; 2.3247x over previous
import functools

import jax
import jax.numpy as jnp
from jax import lax
from jax.experimental import pallas as pl
from jax.experimental.pallas import tpu as pltpu
from jax.experimental.pallas import tpu_sc as plsc

B = 1024
L = 200
EMB = 64
FEMB = 32
OUT_D = EMB + 2 * FEMB

N = B * L
NC = 2
NS = 16
NW = NC * NS
PER_W = N // NW
C = 128
NCH = PER_W // C


@jax.jit
def _wordrep_sc(widx, f0idx, f1idx, wt, f0t, f1t):
    mesh = plsc.VectorSubcoreMesh(core_axis_name="c", subcore_axis_name="s")

    @functools.partial(
        pl.kernel,
        out_type=jax.ShapeDtypeStruct((N, OUT_D), jnp.float32),
        mesh=mesh,
        compiler_params=pltpu.CompilerParams(use_tc_tiling_on_sc=False),
        scratch_types=[
            pltpu.VMEM((NCH, C), jnp.int32),
            pltpu.VMEM((NCH, C), jnp.int32),
            pltpu.VMEM((NCH, C), jnp.int32),
            pltpu.VMEM((C, EMB), jnp.float32),
            pltpu.VMEM((C, FEMB), jnp.float32),
            pltpu.VMEM((C, FEMB), jnp.float32),
            pltpu.SemaphoreType.DMA,
        ],
    )
    def k(widx_hbm, f0idx_hbm, f1idx_hbm, wt_hbm, f0t_hbm, f1t_hbm,
          out_hbm, widx_v, f0idx_v, f1idx_v, wrows, f0rows, f1rows, gsem):
        wid = lax.axis_index("s") * NC + lax.axis_index("c")
        row0 = wid * NCH
        pltpu.sync_copy(widx_hbm.at[pl.ds(row0, NCH)], widx_v)
        pltpu.sync_copy(f0idx_hbm.at[pl.ds(row0, NCH)], f0idx_v)
        pltpu.sync_copy(f1idx_hbm.at[pl.ds(row0, NCH)], f1idx_v)

        def body(j, _):
            cw = pltpu.async_copy(wt_hbm.at[widx_v.at[j]], wrows, gsem)
            c0 = pltpu.async_copy(f0t_hbm.at[f0idx_v.at[j]], f0rows, gsem)
            c1 = pltpu.async_copy(f1t_hbm.at[f1idx_v.at[j]], f1rows, gsem)
            cw.wait()
            c0.wait()
            c1.wait()
            base = wid * PER_W + j * C
            pltpu.sync_copy(wrows, out_hbm.at[pl.ds(base, C), pl.ds(0, EMB)])
            pltpu.sync_copy(f0rows, out_hbm.at[pl.ds(base, C), pl.ds(EMB, FEMB)])
            pltpu.sync_copy(f1rows, out_hbm.at[pl.ds(base, C), pl.ds(EMB + FEMB, FEMB)])
            return 0

        lax.fori_loop(0, NCH, body, 0)

    return k(widx, f0idx, f1idx, wt, f0t, f1t)


def kernel(word_inputs, feature_input_0, feature_input_1,
           word_emb_table, feat_table_0, feat_table_1):
    widx = jnp.asarray(word_inputs, jnp.int32).reshape(NW * NCH, C)
    f0idx = jnp.asarray(feature_input_0, jnp.int32).reshape(NW * NCH, C)
    f1idx = jnp.asarray(feature_input_1, jnp.int32).reshape(NW * NCH, C)
    out = _wordrep_sc(widx, f0idx, f1idx,
                      word_emb_table, feat_table_0, feat_table_1)
    return out.reshape(B, L, OUT_D)


# speedup vs baseline: 2.4068x; 1.0353x over previous
import functools

import jax
import jax.numpy as jnp
from jax import lax
from jax.experimental import pallas as pl
from jax.experimental.pallas import tpu as pltpu
from jax.experimental.pallas import tpu_sc as plsc

B = 1024
L = 200
EMB = 64
FEMB = 32
OUT_D = EMB + 2 * FEMB

N = B * L
NC = 2
NS = 16
NW = NC * NS
PER_W = N // NW
C = 128
NCH = PER_W // C


@jax.jit
def _wordrep_sc(widx, f0idx, f1idx, wt, f0t, f1t):
    mesh = plsc.VectorSubcoreMesh(core_axis_name="c", subcore_axis_name="s")

    @functools.partial(
        pl.kernel,
        out_type=jax.ShapeDtypeStruct((N, OUT_D), jnp.float32),
        mesh=mesh,
        compiler_params=pltpu.CompilerParams(use_tc_tiling_on_sc=False),
        scratch_types=[
            pltpu.VMEM((NCH, C), jnp.int32),
            pltpu.VMEM((NCH, C), jnp.int32),
            pltpu.VMEM((NCH, C), jnp.int32),
            pltpu.VMEM((2, C, EMB), jnp.float32),
            pltpu.VMEM((2, C, FEMB), jnp.float32),
            pltpu.VMEM((2, C, FEMB), jnp.float32),
            pltpu.SemaphoreType.DMA,
            pltpu.SemaphoreType.DMA,
            pltpu.SemaphoreType.DMA,
            pltpu.SemaphoreType.DMA,
        ],
    )
    def k(widx_hbm, f0idx_hbm, f1idx_hbm, wt_hbm, f0t_hbm, f1t_hbm,
          out_hbm, widx_v, f0idx_v, f1idx_v, wrows, f0rows, f1rows,
          gsem0, gsem1, osem0, osem1):
        wid = lax.axis_index("s") * NC + lax.axis_index("c")
        row0 = wid * NCH
        pltpu.sync_copy(widx_hbm.at[pl.ds(row0, NCH)], widx_v)
        pltpu.sync_copy(f0idx_hbm.at[pl.ds(row0, NCH)], f0idx_v)
        pltpu.sync_copy(f1idx_hbm.at[pl.ds(row0, NCH)], f1idx_v)

        def g_start(j, s, gsem):
            pltpu.async_copy(wt_hbm.at[widx_v.at[j]], wrows.at[s], gsem)
            pltpu.async_copy(f0t_hbm.at[f0idx_v.at[j]], f0rows.at[s], gsem)
            pltpu.async_copy(f1t_hbm.at[f1idx_v.at[j]], f1rows.at[s], gsem)

        def g_wait(j, s, gsem):
            pltpu.make_async_copy(wt_hbm.at[widx_v.at[j]], wrows.at[s], gsem).wait()
            pltpu.make_async_copy(f0t_hbm.at[f0idx_v.at[j]], f0rows.at[s], gsem).wait()
            pltpu.make_async_copy(f1t_hbm.at[f1idx_v.at[j]], f1rows.at[s], gsem).wait()

        def o_start(j, s, osem):
            base = wid * PER_W + j * C
            pltpu.async_copy(wrows.at[s],
                             out_hbm.at[pl.ds(base, C), pl.ds(0, EMB)], osem)
            pltpu.async_copy(f0rows.at[s],
                             out_hbm.at[pl.ds(base, C), pl.ds(EMB, FEMB)], osem)
            pltpu.async_copy(f1rows.at[s],
                             out_hbm.at[pl.ds(base, C), pl.ds(EMB + FEMB, FEMB)], osem)

        def o_wait(s, osem):
            base = wid * PER_W
            pltpu.make_async_copy(wrows.at[s],
                                  out_hbm.at[pl.ds(base, C), pl.ds(0, EMB)], osem).wait()
            pltpu.make_async_copy(f0rows.at[s],
                                  out_hbm.at[pl.ds(base, C), pl.ds(EMB, FEMB)], osem).wait()
            pltpu.make_async_copy(f1rows.at[s],
                                  out_hbm.at[pl.ds(base, C), pl.ds(EMB + FEMB, FEMB)], osem).wait()

        g_start(0, 0, gsem0)
        g_start(1, 1, gsem1)

        def body(i, _):
            a = 2 * i
            g_wait(a, 0, gsem0)
            o_start(a, 0, osem0)
            g_wait(a + 1, 1, gsem1)
            o_start(a + 1, 1, osem1)
            o_wait(0, osem0)
            g_start(a + 2, 0, gsem0)
            o_wait(1, osem1)
            g_start(a + 3, 1, gsem1)
            return 0

        lax.fori_loop(0, (NCH - 2) // 2, body, 0)

        g_wait(NCH - 2, 0, gsem0)
        o_start(NCH - 2, 0, osem0)
        g_wait(NCH - 1, 1, gsem1)
        o_start(NCH - 1, 1, osem1)
        o_wait(0, osem0)
        o_wait(1, osem1)

    return k(widx, f0idx, f1idx, wt, f0t, f1t)


def kernel(word_inputs, feature_input_0, feature_input_1,
           word_emb_table, feat_table_0, feat_table_1):
    widx = jnp.asarray(word_inputs, jnp.int32).reshape(NW * NCH, C)
    f0idx = jnp.asarray(feature_input_0, jnp.int32).reshape(NW * NCH, C)
    f1idx = jnp.asarray(feature_input_1, jnp.int32).reshape(NW * NCH, C)
    out = _wordrep_sc(widx, f0idx, f1idx,
                      word_emb_table, feat_table_0, feat_table_1)
    return out.reshape(B, L, OUT_D)
